# Initial kernel scaffold; baseline (speedup 1.0000x reference)
import functools

import jax
import jax.numpy as jnp
from jax import lax
from jax.experimental import pallas as pl
from jax.experimental.pallas import tpu as pltpu
from jax.experimental.pallas import tpu_sc as plsc

_N_FIELDS = 26
_VOCAB = 100000
_EMB_DIM = 32
_BATCH = 16384
_N_NUM = 13
_BN_EPS = 1e-5

_NC = 2
_NS = 16
_NW = _NC * _NS

_ROWS = _BATCH * _N_FIELDS
_RPW = _ROWS // _NW
_CHUNK = 128
_CPW = _RPW // _CHUNK
_GROUP_CHUNKS = 4
_GROUP_ROWS = _CHUNK * _GROUP_CHUNKS
_N_GROUPS = _RPW // _GROUP_ROWS


def _sc_gather(tables_flat, idx):
    mesh = plsc.VectorSubcoreMesh(
        core_axis_name="c", subcore_axis_name="s",
        num_cores=_NC, num_subcores=_NS)

    @functools.partial(
        pl.kernel,
        out_type=jax.ShapeDtypeStruct((_ROWS, _EMB_DIM), jnp.float32),
        mesh=mesh,
        scratch_types=[
            pltpu.VMEM((_CPW, _CHUNK), jnp.int32),
            pltpu.VMEM((_GROUP_ROWS, _EMB_DIM), jnp.float32),
            pltpu.SemaphoreType.DMA,
        ],
    )
    def k(tbl_hbm, idx_hbm, out_hbm, idx_v, gbuf, sem):
        wid = lax.axis_index("s") * _NC + lax.axis_index("c")
        pltpu.sync_copy(idx_hbm.at[wid], idx_v)
        base_row = wid * _RPW

        def group(g, carry):
            for j in range(_GROUP_CHUNKS):
                pltpu.async_copy(
                    tbl_hbm.at[idx_v.at[g * _GROUP_CHUNKS + j]],
                    gbuf.at[pl.ds(j * _CHUNK, _CHUNK)],
                    sem)
            for j in range(_GROUP_CHUNKS):
                pltpu.make_async_copy(
                    tbl_hbm.at[idx_v.at[g * _GROUP_CHUNKS + j]],
                    gbuf.at[pl.ds(j * _CHUNK, _CHUNK)],
                    sem).wait()
            pltpu.sync_copy(
                gbuf, out_hbm.at[pl.ds(base_row + g * _GROUP_ROWS, _GROUP_ROWS)])
            return carry

        lax.fori_loop(0, _N_GROUPS, group, 0)

    return k(tables_flat, idx)


def _bn_body(xt_ref, g_ref, b_ref, o_ref):
    x = xt_ref[...]
    mean = jnp.mean(x, axis=1, keepdims=True)
    xc = x - mean
    var = jnp.mean(xc * xc, axis=1, keepdims=True)
    o_ref[...] = xc * lax.rsqrt(var + _BN_EPS) * g_ref[...] + b_ref[...]


def kernel(x_numerical, x_cat, tables, gamma, beta):
    idx = (x_cat.astype(jnp.int32)
           + jnp.arange(_N_FIELDS, dtype=jnp.int32) * _VOCAB)
    idx = idx.reshape(_NW, _CPW, _CHUNK)
    tables_flat = tables.reshape(_N_FIELDS * _VOCAB, _EMB_DIM)

    emb = _sc_gather(tables_flat, idx).reshape(_BATCH, _N_FIELDS * _EMB_DIM)

    cont_t = pl.pallas_call(
        _bn_body,
        out_shape=jax.ShapeDtypeStruct((_N_NUM, _BATCH), jnp.float32),
    )(x_numerical.T, gamma.reshape(_N_NUM, 1), beta.reshape(_N_NUM, 1))

    return jnp.concatenate([emb, cont_t.T], axis=1)


# speedup vs baseline: 1.1710x; 1.1710x over previous
import functools

import jax
import jax.numpy as jnp
from jax import lax
from jax.experimental import pallas as pl
from jax.experimental.pallas import tpu as pltpu
from jax.experimental.pallas import tpu_sc as plsc

_N_FIELDS = 26
_VOCAB = 100000
_EMB_DIM = 32
_BATCH = 16384
_N_NUM = 13
_BN_EPS = 1e-5

_NC = 2
_NS = 16
_NW = _NC * _NS

_ROWS = _BATCH * _N_FIELDS
_RPW = _ROWS // _NW
_CHUNK = 128
_CPW = _RPW // _CHUNK
_GROUP_CHUNKS = 4
_GROUP_ROWS = _CHUNK * _GROUP_CHUNKS
_N_GROUPS = _RPW // _GROUP_ROWS


def _sc_gather(tables_flat, idx):
    mesh = plsc.VectorSubcoreMesh(
        core_axis_name="c", subcore_axis_name="s",
        num_cores=_NC, num_subcores=_NS)

    @functools.partial(
        pl.kernel,
        out_type=jax.ShapeDtypeStruct((_ROWS, _EMB_DIM), jnp.float32),
        mesh=mesh,
        scratch_types=[
            pltpu.VMEM((_CPW, _CHUNK), jnp.int32),
            pltpu.VMEM((_GROUP_ROWS, _EMB_DIM), jnp.float32),
            pltpu.SemaphoreType.DMA,
        ],
        compiler_params=pltpu.CompilerParams(use_tc_tiling_on_sc=False),
    )
    def k(tbl_hbm, idx_hbm, out_hbm, idx_v, gbuf, sem):
        wid = lax.axis_index("s") * _NC + lax.axis_index("c")
        pltpu.sync_copy(idx_hbm.at[wid], idx_v)
        base_row = wid * _RPW

        def group(g, carry):
            for j in range(_GROUP_CHUNKS):
                pltpu.async_copy(
                    tbl_hbm.at[idx_v.at[g * _GROUP_CHUNKS + j]],
                    gbuf.at[pl.ds(j * _CHUNK, _CHUNK)],
                    sem)
            for j in range(_GROUP_CHUNKS):
                pltpu.make_async_copy(
                    tbl_hbm.at[idx_v.at[g * _GROUP_CHUNKS + j]],
                    gbuf.at[pl.ds(j * _CHUNK, _CHUNK)],
                    sem).wait()
            pltpu.sync_copy(
                gbuf, out_hbm.at[pl.ds(base_row + g * _GROUP_ROWS, _GROUP_ROWS)])
            return carry

        lax.fori_loop(0, _N_GROUPS, group, 0)

    return k(tables_flat, idx)


def _bn_body(xt_ref, g_ref, b_ref, o_ref):
    x = xt_ref[...]
    mean = jnp.mean(x, axis=1, keepdims=True)
    xc = x - mean
    var = jnp.mean(xc * xc, axis=1, keepdims=True)
    o_ref[...] = xc * lax.rsqrt(var + _BN_EPS) * g_ref[...] + b_ref[...]


def kernel(x_numerical, x_cat, tables, gamma, beta):
    idx = (x_cat.astype(jnp.int32)
           + jnp.arange(_N_FIELDS, dtype=jnp.int32) * _VOCAB)
    idx = idx.reshape(_NW, _CPW, _CHUNK)
    tables_flat = tables.reshape(_N_FIELDS * _VOCAB, _EMB_DIM)

    emb = _sc_gather(tables_flat, idx).reshape(_BATCH, _N_FIELDS * _EMB_DIM)

    cont_t = pl.pallas_call(
        _bn_body,
        out_shape=jax.ShapeDtypeStruct((_N_NUM, _BATCH), jnp.float32),
    )(x_numerical.T, gamma.reshape(_N_NUM, 1), beta.reshape(_N_NUM, 1))

    return jnp.concatenate([emb, cont_t.T], axis=1)
